# baseline (speedup 1.0000x reference)
import functools

import jax
import jax.numpy as jnp
from jax import lax
from jax.experimental import pallas as pl
from jax.experimental.pallas import tpu as pltpu
from jax.experimental.pallas import tpu_sc as plsc

_B = 16384
_F = 32
_RPB = 4
_CHUNK = 128


@functools.cache
def _build(num_rows):
    info = plsc.get_sparse_core_info()
    nc, ns, nl = info.num_cores, info.num_subcores, info.num_lanes
    nw = nc * ns
    bpw = _B // nw
    nch = bpw // _CHUNK
    mesh = plsc.VectorSubcoreMesh(core_axis_name="c", subcore_axis_name="s")

    @functools.partial(
        pl.kernel,
        mesh=mesh,
        out_type=jax.ShapeDtypeStruct((_B,), jnp.float32),
        compiler_params=pltpu.CompilerParams(needs_layout_passes=False),
        scratch_types=[
            pltpu.VMEM((nch, _CHUNK), jnp.int32),
            pltpu.VMEM((nch, _CHUNK), jnp.int32),
            pltpu.VMEM((nch, _CHUNK), jnp.int32),
            pltpu.VMEM((nch, _CHUNK), jnp.int32),
            pltpu.VMEM((_CHUNK, 128), jnp.float32),
            pltpu.VMEM((_CHUNK, 128), jnp.float32),
            pltpu.VMEM((_CHUNK, 128), jnp.float32),
            pltpu.VMEM((_CHUNK, 128), jnp.float32),
            pltpu.VMEM((bpw,), jnp.float32),
            pltpu.SemaphoreType.DMA,
            pltpu.SemaphoreType.DMA,
            pltpu.SemaphoreType.DMA,
            pltpu.SemaphoreType.DMA,
        ],
    )
    def sc_dot(ublk_h, ucol_h, iblk_h, icol_h, utab_h, itab_h, out_h,
               ublk_v, ucol_v, iblk_v, icol_v,
               ubuf0, ubuf1, ibuf0, ibuf1, out_v,
               sem_u0, sem_u1, sem_i0, sem_i1):
        wid = lax.axis_index("s") * nc + lax.axis_index("c")
        base = wid * bpw
        ubufs, ibufs = (ubuf0, ubuf1), (ibuf0, ibuf1)
        sems_u, sems_i = (sem_u0, sem_u1), (sem_i0, sem_i1)

        for j in range(nch):
            sl = pl.ds(base + j * _CHUNK, _CHUNK)
            pltpu.sync_copy(ublk_h.at[sl], ublk_v.at[j])
            pltpu.sync_copy(ucol_h.at[sl], ucol_v.at[j])
            pltpu.sync_copy(iblk_h.at[sl], iblk_v.at[j])
            pltpu.sync_copy(icol_h.at[sl], icol_v.at[j])

        def fire(j):
            b = j % 2
            return (pltpu.async_copy(utab_h.at[ublk_v.at[j]], ubufs[b], sems_u[b]),
                    pltpu.async_copy(itab_h.at[iblk_v.at[j]], ibufs[b], sems_i[b]))

        lanes = lax.iota(jnp.int32, nl)
        pending = fire(0)

        for j in range(nch):
            cu, ci = pending
            if j + 1 < nch:
                nxt = fire(j + 1)
            cu.wait()
            ci.wait()
            ub, ib = ubufs[j % 2], ibufs[j % 2]

            def body(g, carry, j=j, ub=ub, ib=ib):
                lrows = g * nl + lanes
                cb_u = ucol_v[j, pl.ds(g * nl, nl)]
                cb_i = icol_v[j, pl.ds(g * nl, nl)]
                acc = jnp.zeros((nl,), jnp.float32)
                for f in range(_F):
                    u = plsc.load_gather(ub, [lrows, cb_u + f])
                    t = plsc.load_gather(ib, [lrows, cb_i + f])
                    acc = acc + u * t
                out_v[pl.ds(j * _CHUNK + g * nl, nl)] = acc
                return carry

            lax.fori_loop(0, _CHUNK // nl, body, 0)
            if j + 1 < nch:
                pending = nxt

        pltpu.sync_copy(out_v, out_h.at[pl.ds(base, bpw)])

    return sc_dot


def kernel(user_indices, item_indices, user_table, item_table):
    num_rows = user_table.shape[0]
    sc_dot = _build(num_rows)
    ui = user_indices.astype(jnp.int32)
    ii = item_indices.astype(jnp.int32)
    return sc_dot(ui >> 2, (ui & 3) << 5,
                  ii >> 2, (ii & 3) << 5,
                  user_table.reshape(num_rows // _RPB, _F * _RPB),
                  item_table.reshape(num_rows // _RPB, _F * _RPB))


# speedup vs baseline: 3.6713x; 3.6713x over previous
import functools

import jax
import jax.numpy as jnp
from jax import lax
from jax.experimental import pallas as pl
from jax.experimental.pallas import tpu as pltpu
from jax.experimental.pallas import tpu_sc as plsc

_B = 16384
_F = 32
_G = 15


@functools.cache
def _build(num_rows):
    info = plsc.get_sparse_core_info()
    nc, ns, nl = info.num_cores, info.num_subcores, info.num_lanes
    nw = nc * ns
    bpw = _B // nw
    ngrp = (bpw + _G - 1) // _G
    pad = ngrp * _G + nl
    max_c = (num_rows + 127) // 128 - 1
    mesh = plsc.VectorSubcoreMesh(core_axis_name="c", subcore_axis_name="s")

    @functools.partial(
        pl.kernel,
        mesh=mesh,
        out_type=jax.ShapeDtypeStruct((_B,), jnp.float32),
        compiler_params=pltpu.CompilerParams(needs_layout_passes=False),
        scratch_types=[
            pltpu.VMEM((pad,), jnp.int32),
            pltpu.VMEM((pad,), jnp.int32),
            pltpu.VMEM((2 * _G, _F, 128), jnp.float32),
            pltpu.VMEM((pad,), jnp.float32),
            pltpu.SemaphoreType.DMA,
            pltpu.SemaphoreType.DMA,
        ],
    )
    def sc_dot(ut_h, it_h, uidx_h, iidx_h, out_h,
               uidx_v, iidx_v, blk_v, out_v, sem_u, sem_i):
        wid = lax.axis_index("s") * nc + lax.axis_index("c")
        base = wid * bpw
        pltpu.sync_copy(uidx_h.at[pl.ds(base, bpw)], uidx_v.at[pl.ds(0, bpw)])
        pltpu.sync_copy(iidx_h.at[pl.ds(base, bpw)], iidx_v.at[pl.ds(0, bpw)])

        lanes = lax.iota(jnp.int32, nl)
        blk_u = jnp.minimum(2 * lanes, 2 * _G - 2)
        blk_i = jnp.minimum(2 * lanes + 1, 2 * _G - 1)

        def body(g, carry):
            iv_u = uidx_v[pl.ds(g * _G, nl)]
            iv_i = iidx_v[pl.ds(g * _G, nl)]

            copies = []
            for j in range(_G):
                c_u = jnp.clip(iv_u[j] >> 7, 0, max_c)
                c_i = jnp.clip(iv_i[j] >> 7, 0, max_c)
                off_u = pl.multiple_of(c_u * 128, 128)
                off_i = pl.multiple_of(c_i * 128, 128)
                copies.append(pltpu.async_copy(
                    ut_h.at[pl.ds(0, _F), pl.ds(off_u, 128)],
                    blk_v.at[2 * j], sem_u))
                copies.append(pltpu.async_copy(
                    it_h.at[pl.ds(0, _F), pl.ds(off_i, 128)],
                    blk_v.at[2 * j + 1], sem_i))
            for cp in copies:
                cp.wait()

            q_u = jnp.bitwise_and(iv_u, 127)
            q_i = jnp.bitwise_and(iv_i, 127)
            acc = jnp.zeros((nl,), jnp.float32)
            for f in range(_F):
                fv = jnp.full((nl,), f, jnp.int32)
                gu = plsc.load_gather(blk_v, [blk_u, fv, q_u])
                gi = plsc.load_gather(blk_v, [blk_i, fv, q_i])
                acc = acc + gu * gi
            out_v[pl.ds(g * _G, nl)] = acc
            return carry

        lax.fori_loop(0, ngrp, body, 0)
        pltpu.sync_copy(out_v.at[pl.ds(0, bpw)], out_h.at[pl.ds(base, bpw)])

    return sc_dot


def kernel(user_indices, item_indices, user_table, item_table):
    sc_dot = _build(user_table.shape[0])
    return sc_dot(user_table.T, item_table.T,
                  user_indices.astype(jnp.int32),
                  item_indices.astype(jnp.int32))


# speedup vs baseline: 4.1520x; 1.1310x over previous
import functools

import jax
import jax.numpy as jnp
from jax import lax
from jax.experimental import pallas as pl
from jax.experimental.pallas import tpu as pltpu
from jax.experimental.pallas import tpu_sc as plsc

_B = 16384
_F = 32
_G = 7


@functools.cache
def _build(num_rows):
    info = plsc.get_sparse_core_info()
    nc, ns, nl = info.num_cores, info.num_subcores, info.num_lanes
    nw = nc * ns
    bpw = _B // nw
    ngrp = (bpw + _G - 1) // _G
    assert ngrp % 2 == 0
    pad = ngrp * _G + nl
    max_c = (num_rows + 127) // 128 - 1
    mesh = plsc.VectorSubcoreMesh(core_axis_name="c", subcore_axis_name="s")

    @functools.partial(
        pl.kernel,
        mesh=mesh,
        out_type=jax.ShapeDtypeStruct((_B,), jnp.float32),
        compiler_params=pltpu.CompilerParams(needs_layout_passes=False),
        scratch_types=[
            pltpu.VMEM((pad,), jnp.int32),
            pltpu.VMEM((pad,), jnp.int32),
            pltpu.VMEM((2 * _G, _F, 128), jnp.float32),
            pltpu.VMEM((2 * _G, _F, 128), jnp.float32),
            pltpu.VMEM((pad,), jnp.float32),
            pltpu.SemaphoreType.DMA,
            pltpu.SemaphoreType.DMA,
            pltpu.SemaphoreType.DMA,
            pltpu.SemaphoreType.DMA,
        ],
    )
    def sc_dot(ut_h, it_h, uidx_h, iidx_h, out_h,
               uidx_v, iidx_v, buf_a, buf_b, out_v,
               sem_ua, sem_ia, sem_ub, sem_ib):
        wid = lax.axis_index("s") * nc + lax.axis_index("c")
        base = wid * bpw
        pltpu.sync_copy(uidx_h.at[pl.ds(base, bpw)], uidx_v.at[pl.ds(0, bpw)])
        pltpu.sync_copy(iidx_h.at[pl.ds(base, bpw)], iidx_v.at[pl.ds(0, bpw)])

        lanes = lax.iota(jnp.int32, nl)
        blk_u = jnp.minimum(2 * lanes, 2 * _G - 2)
        blk_i = jnp.minimum(2 * lanes + 1, 2 * _G - 1)

        def fire(g, buf, sem_u, sem_i):
            iv_u = uidx_v[pl.ds(g * _G, nl)]
            iv_i = iidx_v[pl.ds(g * _G, nl)]
            for j in range(_G):
                c_u = jnp.clip(iv_u[j] >> 7, 0, max_c)
                c_i = jnp.clip(iv_i[j] >> 7, 0, max_c)
                off_u = pl.multiple_of(c_u * 128, 128)
                off_i = pl.multiple_of(c_i * 128, 128)
                pltpu.async_copy(ut_h.at[pl.ds(0, _F), pl.ds(off_u, 128)],
                                 buf.at[2 * j], sem_u)
                pltpu.async_copy(it_h.at[pl.ds(0, _F), pl.ds(off_i, 128)],
                                 buf.at[2 * j + 1], sem_i)

        def drain_compute(g, buf, sem_u, sem_i):
            for j in range(_G):
                pltpu.make_async_copy(ut_h.at[pl.ds(0, _F), pl.ds(0, 128)],
                                      buf.at[2 * j], sem_u).wait()
                pltpu.make_async_copy(it_h.at[pl.ds(0, _F), pl.ds(0, 128)],
                                      buf.at[2 * j + 1], sem_i).wait()
            iv_u = uidx_v[pl.ds(g * _G, nl)]
            iv_i = iidx_v[pl.ds(g * _G, nl)]
            q_u = jnp.bitwise_and(iv_u, 127)
            q_i = jnp.bitwise_and(iv_i, 127)
            acc = jnp.zeros((nl,), jnp.float32)
            for f in range(_F):
                fv = jnp.full((nl,), f, jnp.int32)
                gu = plsc.load_gather(buf, [blk_u, fv, q_u])
                gi = plsc.load_gather(buf, [blk_i, fv, q_i])
                acc = acc + gu * gi
            out_v[pl.ds(g * _G, nl)] = acc

        fire(0, buf_a, sem_ua, sem_ia)

        def body(p, carry):
            g = 2 * p
            fire(g + 1, buf_b, sem_ub, sem_ib)
            drain_compute(g, buf_a, sem_ua, sem_ia)

            @pl.when(p < ngrp // 2 - 1)
            def _():
                fire(g + 2, buf_a, sem_ua, sem_ia)

            drain_compute(g + 1, buf_b, sem_ub, sem_ib)
            return carry

        lax.fori_loop(0, ngrp // 2, body, 0)
        pltpu.sync_copy(out_v.at[pl.ds(0, bpw)], out_h.at[pl.ds(base, bpw)])

    return sc_dot


def kernel(user_indices, item_indices, user_table, item_table):
    sc_dot = _build(user_table.shape[0])
    return sc_dot(user_table.T, item_table.T,
                  user_indices.astype(jnp.int32),
                  item_indices.astype(jnp.int32))


# speedup vs baseline: 4.3985x; 1.0594x over previous
import functools

import jax
import jax.numpy as jnp
from jax import lax
from jax.experimental import pallas as pl
from jax.experimental.pallas import tpu as pltpu
from jax.experimental.pallas import tpu_sc as plsc

_B = 16384
_F = 32
_G = 3
_NBUF = 4


@functools.cache
def _build(num_rows):
    info = plsc.get_sparse_core_info()
    nc, ns, nl = info.num_cores, info.num_subcores, info.num_lanes
    nw = nc * ns
    bpw = _B // nw
    ngrp = (bpw + _G - 1) // _G
    ngrp = ((ngrp + _NBUF - 1) // _NBUF) * _NBUF
    pad = ngrp * _G + nl
    max_c = (num_rows + 127) // 128 - 1
    mesh = plsc.VectorSubcoreMesh(core_axis_name="c", subcore_axis_name="s")

    @functools.partial(
        pl.kernel,
        mesh=mesh,
        out_type=jax.ShapeDtypeStruct((_B,), jnp.float32),
        compiler_params=pltpu.CompilerParams(needs_layout_passes=False),
        scratch_types=[
            pltpu.VMEM((pad,), jnp.int32),
            pltpu.VMEM((pad,), jnp.int32),
            pltpu.VMEM((2 * _G, _F, 128), jnp.float32),
            pltpu.VMEM((2 * _G, _F, 128), jnp.float32),
            pltpu.VMEM((2 * _G, _F, 128), jnp.float32),
            pltpu.VMEM((2 * _G, _F, 128), jnp.float32),
            pltpu.VMEM((pad,), jnp.float32),
            pltpu.SemaphoreType.DMA,
            pltpu.SemaphoreType.DMA,
            pltpu.SemaphoreType.DMA,
            pltpu.SemaphoreType.DMA,
            pltpu.SemaphoreType.DMA,
            pltpu.SemaphoreType.DMA,
            pltpu.SemaphoreType.DMA,
            pltpu.SemaphoreType.DMA,
        ],
    )
    def sc_dot(ut_h, it_h, uidx_h, iidx_h, out_h,
               uidx_v, iidx_v, b0, b1, b2, b3, out_v,
               su0, si0, su1, si1, su2, si2, su3, si3):
        wid = lax.axis_index("s") * nc + lax.axis_index("c")
        base = wid * bpw
        pltpu.sync_copy(uidx_h.at[pl.ds(base, bpw)], uidx_v.at[pl.ds(0, bpw)])
        pltpu.sync_copy(iidx_h.at[pl.ds(base, bpw)], iidx_v.at[pl.ds(0, bpw)])

        bufs = (b0, b1, b2, b3)
        sems = ((su0, si0), (su1, si1), (su2, si2), (su3, si3))
        lanes = lax.iota(jnp.int32, nl)
        blk_u = jnp.minimum(2 * lanes, 2 * _G - 2)
        blk_i = jnp.minimum(2 * lanes + 1, 2 * _G - 1)

        def fire(g, b):
            buf, (sem_u, sem_i) = bufs[b], sems[b]
            iv_u = uidx_v[pl.ds(g * _G, nl)]
            iv_i = iidx_v[pl.ds(g * _G, nl)]
            for j in range(_G):
                c_u = jnp.clip(iv_u[j] >> 7, 0, max_c)
                c_i = jnp.clip(iv_i[j] >> 7, 0, max_c)
                off_u = pl.multiple_of(c_u * 128, 128)
                off_i = pl.multiple_of(c_i * 128, 128)
                pltpu.async_copy(ut_h.at[pl.ds(0, _F), pl.ds(off_u, 128)],
                                 buf.at[2 * j], sem_u)
                pltpu.async_copy(it_h.at[pl.ds(0, _F), pl.ds(off_i, 128)],
                                 buf.at[2 * j + 1], sem_i)

        def drain_compute(g, b):
            buf, (sem_u, sem_i) = bufs[b], sems[b]
            for j in range(_G):
                pltpu.make_async_copy(ut_h.at[pl.ds(0, _F), pl.ds(0, 128)],
                                      buf.at[2 * j], sem_u).wait()
                pltpu.make_async_copy(it_h.at[pl.ds(0, _F), pl.ds(0, 128)],
                                      buf.at[2 * j + 1], sem_i).wait()
            iv_u = uidx_v[pl.ds(g * _G, nl)]
            iv_i = iidx_v[pl.ds(g * _G, nl)]
            q_u = jnp.bitwise_and(iv_u, 127)
            q_i = jnp.bitwise_and(iv_i, 127)
            acc = jnp.zeros((nl,), jnp.float32)
            for f in range(_F):
                fv = jnp.full((nl,), f, jnp.int32)
                gu = plsc.load_gather(buf, [blk_u, fv, q_u])
                gi = plsc.load_gather(buf, [blk_i, fv, q_i])
                acc = acc + gu * gi
            out_v[pl.ds(g * _G, nl)] = acc

        for b in range(_NBUF - 1):
            fire(b, b)

        def body(p, carry):
            g0 = _NBUF * p
            for r in range(_NBUF):
                g = g0 + r

                @pl.when(g + _NBUF - 1 < ngrp)
                def _(g=g, r=r):
                    fire(g + _NBUF - 1, (r + _NBUF - 1) % _NBUF)

                drain_compute(g, r)
            return carry

        lax.fori_loop(0, ngrp // _NBUF, body, 0)
        pltpu.sync_copy(out_v.at[pl.ds(0, bpw)], out_h.at[pl.ds(base, bpw)])

    return sc_dot


def kernel(user_indices, item_indices, user_table, item_table):
    sc_dot = _build(user_table.shape[0])
    return sc_dot(user_table.T, item_table.T,
                  user_indices.astype(jnp.int32),
                  item_indices.astype(jnp.int32))
